# speedup vs baseline: 6.8560x; 6.8560x over previous
import functools

import jax
import jax.numpy as jnp
from jax import lax
from jax.experimental import pallas as pl
from jax.experimental.pallas import tpu as pltpu
from jax.experimental.pallas import tpu_sc as plsc

N_NODES = 10000
D = 128
N_EDGES = 320000
BN_EPS = 1e-5

NC = 2
NS = 16
NW = NC * NS
EW = N_EDGES // NW
CK = 80
CH = EW // CK
NPAD = 10240
RT = NPAD // NS


def _sc_aggregate(x, col_w, row_w):
    mesh = plsc.VectorSubcoreMesh(core_axis_name="c", subcore_axis_name="s")

    @functools.partial(
        pl.kernel,
        out_type=jax.ShapeDtypeStruct((NC, NPAD, D), jnp.float32),
        mesh=mesh,
        scratch_types=[
            pltpu.VMEM_SHARED((NPAD, D), jnp.float32),
            pltpu.VMEM((CH, CK), jnp.int32),
            pltpu.VMEM((CH, CK), jnp.int32),
            pltpu.VMEM((CK, D), jnp.float32),
            pltpu.SemaphoreType.DMA,
        ],
    )
    def agg_kernel(x_hbm, col_hbm, row_hbm, out_hbm, acc, cidx, ridx, rows, sem):
        core = lax.axis_index("c")
        sid = lax.axis_index("s")
        wid = sid * NC + core

        def _zfill(k, carry):
            rows[k // (D // 16), pl.ds((k % (D // 16)) * 16, 16)] = (
                jnp.zeros((16,), jnp.float32))
            return carry
        lax.fori_loop(0, CK * (D // 16), _zfill, 0)

        def _zcopy(b, carry):
            pltpu.sync_copy(rows, acc.at[pl.ds(sid * RT + b * CK, CK)])
            return carry
        lax.fori_loop(0, RT // CK, _zcopy, 0)
        plsc.subcore_barrier()

        pltpu.sync_copy(col_hbm.at[wid], cidx)
        pltpu.sync_copy(row_hbm.at[wid], ridx)

        def _chunk(c, carry):
            pltpu.async_copy(x_hbm.at[cidx.at[c]], rows, sem).wait()
            pltpu.sync_copy(rows, acc.at[ridx.at[c]], add=True)
            return carry
        lax.fori_loop(0, CH, _chunk, 0)
        plsc.subcore_barrier()

        pltpu.sync_copy(acc.at[pl.ds(sid * RT, RT)],
                        out_hbm.at[core, pl.ds(sid * RT, RT)])

    return agg_kernel(x, col_w, row_w)


def _mlp_body(x_ref, p_ref, eps_ref, w1_ref, b1_ref, g1_ref, be1_ref,
              w2_ref, b2_ref, g2_ref, be2_ref, o_ref):
    agg = p_ref[0, :N_NODES, :] + p_ref[1, :N_NODES, :]
    out = (1.0 + eps_ref[0]) * x_ref[...] + agg
    h = lax.dot_general(out, w1_ref[...], (((1,), (1,)), ((), ())),
                        preferred_element_type=jnp.float32)
    h = h + b1_ref[...]
    mean = jnp.mean(h, axis=0, keepdims=True)
    var = jnp.mean((h - mean) ** 2, axis=0, keepdims=True)
    h = (h - mean) / jnp.sqrt(var + BN_EPS) * g1_ref[...] + be1_ref[...]
    h = jnp.maximum(h, 0.0)
    h = lax.dot_general(h, w2_ref[...], (((1,), (1,)), ((), ())),
                        preferred_element_type=jnp.float32)
    h = h + b2_ref[...]
    mean = jnp.mean(h, axis=0, keepdims=True)
    var = jnp.mean((h - mean) ** 2, axis=0, keepdims=True)
    h = (h - mean) / jnp.sqrt(var + BN_EPS) * g2_ref[...] + be2_ref[...]
    o_ref[...] = jnp.maximum(h, 0.0)


def _mlp(x, partials, eps, W1, b1, g1, be1, W2, b2, g2, be2):
    vmem = pl.BlockSpec(memory_space=pltpu.VMEM)
    smem = pl.BlockSpec(memory_space=pltpu.SMEM)
    return pl.pallas_call(
        _mlp_body,
        out_shape=jax.ShapeDtypeStruct((N_NODES, D), jnp.float32),
        in_specs=[vmem, vmem, smem] + [vmem] * 8,
        out_specs=vmem,
    )(x, partials, eps, W1, b1.reshape(1, D), g1.reshape(1, D),
      be1.reshape(1, D), W2, b2.reshape(1, D), g2.reshape(1, D),
      be2.reshape(1, D))


def kernel(x, edge_index, eps, W1, b1, g1, be1, W2, b2, g2, be2):
    row = edge_index[0].astype(jnp.int32).reshape(NW, CH, CK)
    col = edge_index[1].astype(jnp.int32).reshape(NW, CH, CK)
    partials = _sc_aggregate(x, col, row)
    return _mlp(x, partials, eps, W1, b1, g1, be1, W2, b2, g2, be2)


# speedup vs baseline: 10.5347x; 1.5366x over previous
import functools

import jax
import jax.numpy as jnp
from jax import lax
from jax.experimental import pallas as pl
from jax.experimental.pallas import tpu as pltpu
from jax.experimental.pallas import tpu_sc as plsc

N_NODES = 10000
D = 128
N_EDGES = 320000
BN_EPS = 1e-5

NC = 2
NS = 16
NW = NC * NS
EW = N_EDGES // NW
CK = 80
CH = EW // CK
NBUF = 2
NPAD = 10240
RT = NPAD // NS


def _sc_aggregate(x, col_w, row_w):
    mesh = plsc.VectorSubcoreMesh(core_axis_name="c", subcore_axis_name="s")

    @functools.partial(
        pl.kernel,
        out_type=jax.ShapeDtypeStruct((NC, NPAD, D), jnp.float32),
        mesh=mesh,
        scratch_types=[
            pltpu.VMEM_SHARED((NPAD, D), jnp.float32),
            pltpu.VMEM((EW,), jnp.int32),
            pltpu.VMEM((CH, CK), jnp.int32),
            pltpu.VMEM((NBUF, CK, D), jnp.float32),
            [pltpu.SemaphoreType.DMA] * NBUF,
            [pltpu.SemaphoreType.DMA] * NBUF,
        ],
    )
    def agg_kernel(x_hbm, col_hbm, row_hbm, out_hbm, acc, cidx, ridx, rows,
                   gsems, ssems):
        core = lax.axis_index("c")
        sid = lax.axis_index("s")
        wid = sid * NC + core

        def _zfill(k, carry):
            rows[0, k // (D // 16), pl.ds((k % (D // 16)) * 16, 16)] = (
                jnp.zeros((16,), jnp.float32))
            return carry
        lax.fori_loop(0, CK * (D // 16), _zfill, 0)

        def _zcopy(b, carry):
            pltpu.sync_copy(rows.at[0], acc.at[pl.ds(sid * RT + b * CK, CK)])
            return carry
        lax.fori_loop(0, RT // CK, _zcopy, 0)
        plsc.subcore_barrier()

        pltpu.sync_copy(col_hbm.at[wid], cidx)
        pltpu.sync_copy(row_hbm.at[wid], ridx)

        for b in range(NBUF):
            pltpu.async_copy(x_hbm.at[cidx.at[pl.ds(b * CK, CK)]],
                             rows.at[b], gsems[b])

        def _round(i, carry):
            for b in range(NBUF):
                c = i * NBUF + b
                pltpu.make_async_copy(x_hbm.at[pl.ds(0, CK)], rows.at[b],
                                      gsems[b]).wait()
                pltpu.sync_copy(rows.at[b], acc.at[ridx.at[c]], add=True)

                @pl.when(c + NBUF < CH)
                def _():
                    pltpu.async_copy(
                        x_hbm.at[cidx.at[pl.ds((c + NBUF) * CK, CK)]],
                        rows.at[b], gsems[b])
            return carry
        lax.fori_loop(0, CH // NBUF, _round, 0)
        pltpu.make_async_copy(x_hbm.at[pl.ds(0, CK)], rows.at[0],
                              gsems[0]).wait()
        pltpu.sync_copy(rows.at[0], acc.at[ridx.at[CH - 1]], add=True)
        plsc.subcore_barrier()

        pltpu.sync_copy(acc.at[pl.ds(sid * RT, RT)],
                        out_hbm.at[core, pl.ds(sid * RT, RT)])

    return agg_kernel(x, col_w, row_w)


def _mlp_body(x_ref, p_ref, eps_ref, w1_ref, b1_ref, g1_ref, be1_ref,
              w2_ref, b2_ref, g2_ref, be2_ref, o_ref):
    agg = p_ref[0, :N_NODES, :] + p_ref[1, :N_NODES, :]
    out = (1.0 + eps_ref[0]) * x_ref[...] + agg
    h = lax.dot_general(out, w1_ref[...], (((1,), (1,)), ((), ())),
                        preferred_element_type=jnp.float32)
    h = h + b1_ref[...]
    mean = jnp.mean(h, axis=0, keepdims=True)
    var = jnp.mean((h - mean) ** 2, axis=0, keepdims=True)
    h = (h - mean) / jnp.sqrt(var + BN_EPS) * g1_ref[...] + be1_ref[...]
    h = jnp.maximum(h, 0.0)
    h = lax.dot_general(h, w2_ref[...], (((1,), (1,)), ((), ())),
                        preferred_element_type=jnp.float32)
    h = h + b2_ref[...]
    mean = jnp.mean(h, axis=0, keepdims=True)
    var = jnp.mean((h - mean) ** 2, axis=0, keepdims=True)
    h = (h - mean) / jnp.sqrt(var + BN_EPS) * g2_ref[...] + be2_ref[...]
    o_ref[...] = jnp.maximum(h, 0.0)


def _mlp(x, partials, eps, W1, b1, g1, be1, W2, b2, g2, be2):
    vmem = pl.BlockSpec(memory_space=pltpu.VMEM)
    smem = pl.BlockSpec(memory_space=pltpu.SMEM)
    return pl.pallas_call(
        _mlp_body,
        out_shape=jax.ShapeDtypeStruct((N_NODES, D), jnp.float32),
        in_specs=[vmem, vmem, smem] + [vmem] * 8,
        out_specs=vmem,
    )(x, partials, eps, W1, b1.reshape(1, D), g1.reshape(1, D),
      be1.reshape(1, D), W2, b2.reshape(1, D), g2.reshape(1, D),
      be2.reshape(1, D))


def kernel(x, edge_index, eps, W1, b1, g1, be1, W2, b2, g2, be2):
    row = edge_index[0].astype(jnp.int32).reshape(NW, CH, CK)
    col = edge_index[1].astype(jnp.int32).reshape(NW, EW)
    partials = _sc_aggregate(x, col, row)
    return _mlp(x, partials, eps, W1, b1, g1, be1, W2, b2, g2, be2)
